# baseline (speedup 1.0000x reference)
import functools

import jax
import jax.numpy as jnp
from jax import lax
from jax.experimental import pallas as pl
from jax.experimental.pallas import tpu as pltpu
from jax.experimental.pallas import tpu_sc as plsc


_ITEMS_PER_BLOCK = 65536


def _score_body(t_ref, w_ref, s_ref):
    s_ref[...] = jnp.sum(t_ref[...] * w_ref[...], axis=0)


def _scores(table, W):
    n, h = table.shape
    tt = jnp.swapaxes(table, 0, 1)
    grid = -(-n // _ITEMS_PER_BLOCK)
    return pl.pallas_call(
        _score_body,
        grid=(grid,),
        in_specs=[
            pl.BlockSpec((h, _ITEMS_PER_BLOCK), lambda i: (0, i)),
            pl.BlockSpec((h, 1), lambda i: (0, 0)),
        ],
        out_specs=pl.BlockSpec((_ITEMS_PER_BLOCK,), lambda i: (i,)),
        out_shape=jax.ShapeDtypeStruct((n,), jnp.float32),
    )(tt, W)


@functools.lru_cache(maxsize=None)
def _make_gather(bsz: int, hist: int):
    info = plsc.get_sparse_core_info()
    nc, ns, lanes = info.num_cores, info.num_subcores, info.num_lanes
    nw = nc * ns
    assert bsz % nw == 0
    rows_w = bsz // nw
    n_per_w = rows_w * hist
    assert n_per_w % lanes == 0

    mesh = plsc.VectorSubcoreMesh(core_axis_name="c", subcore_axis_name="s")

    @functools.partial(
        pl.kernel,
        mesh=mesh,
        out_type=jax.ShapeDtypeStruct((bsz * hist,), jnp.float32),
        scratch_types=[
            pltpu.VMEM((n_per_w,), jnp.int32),
            pltpu.VMEM((n_per_w,), jnp.float32),
            pltpu.SemaphoreType.DMA,
        ],
    )
    def gather_k(s_hbm, idx_hbm, out_hbm, idx_v, val_v, sem):
        wid = lax.axis_index("s") * nc + lax.axis_index("c")
        base = wid * n_per_w
        pltpu.sync_copy(idx_hbm.at[pl.ds(base, n_per_w)], idx_v)
        pltpu.async_copy(s_hbm.at[idx_v], val_v, sem).wait()
        pltpu.sync_copy(val_v, out_hbm.at[pl.ds(base, n_per_w)])

    return gather_k


@functools.lru_cache(maxsize=None)
def _make_flatten(bsz: int, hist: int):
    info = plsc.get_sparse_core_info()
    nc, ns = info.num_cores, info.num_subcores
    nw = nc * ns
    assert bsz % (8 * nw) == 0
    cols_w = bsz // nw

    mesh = plsc.VectorSubcoreMesh(core_axis_name="c", subcore_axis_name="s")

    @functools.partial(
        pl.kernel,
        mesh=mesh,
        out_type=jax.ShapeDtypeStruct((bsz * hist,), jnp.int32),
        scratch_types=[pltpu.SemaphoreType.DMA],
    )
    def flat_k(xt_hbm, out_hbm, sem):
        wid = lax.axis_index("s") * nc + lax.axis_index("c")
        c0 = wid * cols_w
        for r in range(hist):
            pltpu.async_copy(
                xt_hbm.at[r, pl.ds(c0, cols_w)],
                out_hbm.at[pl.ds(r * bsz + c0, cols_w)],
                sem,
            )
        for r in range(hist):
            pltpu.make_async_copy(
                xt_hbm.at[r, pl.ds(c0, cols_w)],
                out_hbm.at[pl.ds(r * bsz + c0, cols_w)],
                sem,
            ).wait()

    return flat_k


def kernel(x, table, W, b):
    bsz, hist = x.shape
    s = _scores(table, W)
    idx_t = _make_flatten(bsz, hist)(jnp.swapaxes(x, 0, 1).astype(jnp.int32))
    y1t = _make_gather(bsz, hist)(s, idx_t)
    return jnp.swapaxes(y1t.reshape(hist, bsz), 0, 1)[..., None] + b[0]


# speedup vs baseline: 1.1262x; 1.1262x over previous
import functools

import jax
import jax.numpy as jnp
from jax import lax
from jax.experimental import pallas as pl
from jax.experimental.pallas import tpu as pltpu
from jax.experimental.pallas import tpu_sc as plsc


_ITEMS_PER_BLOCK = 65536


def _score_body(t_ref, w_ref, s_ref):
    s_ref[...] = jnp.sum(t_ref[...] * w_ref[...], axis=0)


def _scores(table, W):
    n, h = table.shape
    tt = jnp.swapaxes(table, 0, 1)
    grid = -(-n // _ITEMS_PER_BLOCK)
    return pl.pallas_call(
        _score_body,
        grid=(grid,),
        in_specs=[
            pl.BlockSpec((h, _ITEMS_PER_BLOCK), lambda i: (0, i)),
            pl.BlockSpec((h, 1), lambda i: (0, 0)),
        ],
        out_specs=pl.BlockSpec((_ITEMS_PER_BLOCK,), lambda i: (i,)),
        out_shape=jax.ShapeDtypeStruct((n,), jnp.float32),
    )(tt, W)


@functools.lru_cache(maxsize=None)
def _make_gather(bsz: int, hist: int):
    info = plsc.get_sparse_core_info()
    nc, ns, lanes = info.num_cores, info.num_subcores, info.num_lanes
    nw = nc * ns
    assert bsz % nw == 0
    rows_w = bsz // nw
    n_per_w = rows_w * hist
    assert n_per_w % lanes == 0

    mesh = plsc.VectorSubcoreMesh(core_axis_name="c", subcore_axis_name="s")

    @functools.partial(
        pl.kernel,
        mesh=mesh,
        out_type=jax.ShapeDtypeStruct((bsz * hist,), jnp.float32),
        scratch_types=[
            pltpu.VMEM((n_per_w,), jnp.int32),
            pltpu.VMEM((n_per_w,), jnp.float32),
            pltpu.SemaphoreType.DMA,
        ],
    )
    def gather_k(s_hbm, idx_hbm, out_hbm, idx_v, val_v, sem):
        wid = lax.axis_index("s") * nc + lax.axis_index("c")
        base = wid * n_per_w
        pltpu.sync_copy(idx_hbm.at[pl.ds(base, n_per_w)], idx_v)
        pltpu.async_copy(s_hbm.at[idx_v], val_v, sem).wait()
        pltpu.sync_copy(val_v, out_hbm.at[pl.ds(base, n_per_w)])

    return gather_k


def kernel(x, table, W, b):
    bsz, hist = x.shape
    s = _scores(table, W)
    idx_t = jnp.minimum(
        jnp.swapaxes(x, 0, 1).reshape(-1).astype(jnp.int32), table.shape[0] - 1
    )
    y1t = _make_gather(bsz, hist)(s, idx_t)
    return jnp.swapaxes(y1t.reshape(hist, bsz), 0, 1)[..., None] + b[0]
